# baseline (speedup 1.0000x reference)
import functools

import jax
import jax.numpy as jnp
from jax import lax
from jax.experimental import pallas as pl
from jax.experimental.pallas import tpu as pltpu
from jax.experimental.pallas import tpu_sc as plsc

_NC = 2
_NS = 16
_NW = _NC * _NS
_CHUNK = 128
_BV = 8192
_HV = _BV // 2
_SC_BLOCKS = 48
_SUBS_PW = 4 * _SC_BLOCKS // _NW
_SUB = 128


def _sc_relayout(embT):
    d = embT.shape[0]
    rows = _SC_BLOCKS * _HV
    n_rounds = _SUBS_PW * (1024 // _SUB)
    mesh = plsc.VectorSubcoreMesh(core_axis_name="c", subcore_axis_name="s")

    def col_range(wid, r):
        unit = wid * _SUBS_PW + r // (1024 // _SUB)
        sc = r % (1024 // _SUB)
        blk = unit // 4
        off = (unit % 4) * 1024
        c0 = pl.multiple_of(blk * _BV + off + sc * _SUB, 128)
        r0 = pl.multiple_of(blk * _HV + off + sc * _SUB, 8)
        return c0, r0

    @functools.partial(
        pl.kernel,
        mesh=mesh,
        out_type=jax.ShapeDtypeStruct((rows, 2 * d), jnp.float32),
        scratch_types=[
            pltpu.VMEM((2 * d, _SUB), jnp.float32),
            pltpu.VMEM((2 * d, _SUB), jnp.float32),
            pltpu.VMEM((2 * _SUB, 2 * d), jnp.float32),
            pltpu.SemaphoreType.DMA,
            pltpu.SemaphoreType.DMA,
            pltpu.SemaphoreType.DMA,
            pltpu.SemaphoreType.DMA,
        ],
        compiler_params=pltpu.CompilerParams(needs_layout_passes=False),
    )
    def relayout_kernel(embT_hbm, out_hbm, lo_v, hi_v, rows_v,
                        sem0, sem1, wsem0, wsem1):
        wid = lax.axis_index("s") * _NC + lax.axis_index("c")
        lane = lax.iota(jnp.int32, 16)
        sems = (sem0, sem1)
        wsems = (wsem0, wsem1)

        def fire(r, buf):
            c0, _ = col_range(wid, r)
            pltpu.async_copy(
                embT_hbm.at[:, pl.ds(c0, _SUB)],
                lo_v.at[pl.ds(buf * d, d)], sems[buf])
            pltpu.async_copy(
                embT_hbm.at[:, pl.ds(c0 + _HV, _SUB)],
                hi_v.at[pl.ds(buf * d, d)], sems[buf])

        def drain(buf):
            pltpu.make_async_copy(
                embT_hbm.at[:, pl.ds(0, _SUB)],
                lo_v.at[pl.ds(buf * d, d)], sems[buf]).wait()
            pltpu.make_async_copy(
                embT_hbm.at[:, pl.ds(0, _SUB)],
                hi_v.at[pl.ds(buf * d, d)], sems[buf]).wait()

        fire(0, 0)

        def round_body(i, _):
            for buf in range(2):
                r = i * 2 + buf
                drain(buf)
                fire(r + 1, 1 - buf)

                def col_body(c, _):
                    cf = jnp.full((16,), c, jnp.int32)
                    rf = jnp.full((16,), buf * _SUB + c, jnp.int32)
                    for q in range(d // 16):
                        plsc.store_scatter(
                            rows_v, [rf, q * 16 + lane],
                            plsc.load_gather(
                                lo_v, [buf * d + q * 16 + lane, cf]))
                        plsc.store_scatter(
                            rows_v, [rf, d + q * 16 + lane],
                            plsc.load_gather(
                                hi_v, [buf * d + q * 16 + lane, cf]))
                    return 0

                lax.fori_loop(0, _SUB, col_body, 0)
                _, r0 = col_range(wid, r)
                pltpu.async_copy(
                    rows_v.at[pl.ds(buf * _SUB, _SUB)],
                    out_hbm.at[pl.ds(r0, _SUB)], wsems[buf]).wait()
            return 0

        lax.fori_loop(0, n_rounds // 2, round_body, 0)
        drain(n_rounds % 2)

    return relayout_kernel(embT)


def _tc_relayout(embT, first_block, last_block=None):
    d, v = embT.shape
    n_blocks = (-(-v // _BV) if last_block is None else last_block) - first_block

    def body(in_ref, out_ref):
        x = in_ref[...]
        out_ref[:, :d] = jnp.transpose(x[:, :_HV], (1, 0))
        out_ref[:, d:] = jnp.transpose(x[:, _HV:], (1, 0))

    return pl.pallas_call(
        body,
        grid=(n_blocks,),
        in_specs=[pl.BlockSpec((d, _BV), lambda i: (0, i + first_block))],
        out_specs=pl.BlockSpec((_HV, 2 * d), lambda i: (i, 0)),
        out_shape=jax.ShapeDtypeStruct((n_blocks * _HV, 2 * d), jnp.float32),
    )(embT)


def _sc_gather(idx2d, table_a, table_b, n_chunks):
    m = idx2d.shape[0] * idx2d.shape[1]
    d2 = table_a.shape[1]
    rows_a = table_a.shape[0]
    rows_b = table_b.shape[0]
    b_per_w = n_chunks * _CHUNK
    mesh = plsc.VectorSubcoreMesh(core_axis_name="c", subcore_axis_name="s")

    @functools.partial(
        pl.kernel,
        mesh=mesh,
        out_type=jax.ShapeDtypeStruct((m, d2), jnp.float32),
        scratch_types=[
            pltpu.VMEM((n_chunks, _CHUNK), jnp.int32),
            pltpu.VMEM((_CHUNK,), jnp.int32),
            pltpu.VMEM((_CHUNK,), jnp.int32),
            pltpu.VMEM((_CHUNK, d2), jnp.float32),
            pltpu.VMEM((_CHUNK, d2), jnp.float32),
            pltpu.VMEM((b_per_w, d2), jnp.float32),
            pltpu.SemaphoreType.DMA,
            pltpu.SemaphoreType.DMA,
        ],
    )
    def gather_kernel(idx_hbm, ta_hbm, tb_hbm, out_hbm,
                      idx_v, aidx_v, bidx_v, sa_v, sb_v, rows_v, sem, osem):
        wid = lax.axis_index("s") * _NC + lax.axis_index("c")
        pltpu.sync_copy(idx_hbm.at[pl.ds(wid * n_chunks, n_chunks)], idx_v)
        zero = jnp.zeros((16,), jnp.int32)
        one = jnp.full((16,), 1, jnp.int32)
        ra = jnp.full((16,), rows_a - 1, jnp.int32)
        rb = jnp.full((16,), rows_b - 1, jnp.int32)

        def chunk_body(g, _):
            def clamp_body(h, _):
                vec = idx_v[g, pl.ds(h * 16, 16)]
                aidx_v[pl.ds(h * 16, 16)] = jnp.minimum(vec, ra)
                bidx_v[pl.ds(h * 16, 16)] = jnp.clip(vec - rows_a, zero, rb)
                return 0

            lax.fori_loop(0, _CHUNK // 16, clamp_body, 0)
            ca = pltpu.async_copy(ta_hbm.at[aidx_v], sa_v, sem)
            cb = pltpu.async_copy(tb_hbm.at[bidx_v], sb_v, sem)
            ca.wait()
            cb.wait()

            def sel_body(h, _):
                vec = idx_v[g, pl.ds(h * 16, 16)]
                for j in range(16):
                    sf = jnp.full((16,), vec[j], jnp.int32)
                    msk = jnp.clip(sf - (rows_a - 1), zero, one).astype(
                        jnp.float32)
                    inv = 1.0 - msk
                    row = g * _CHUNK + h * 16 + j
                    for q in range(d2 // 16):
                        av = sa_v[h * 16 + j, pl.ds(q * 16, 16)]
                        bv = sb_v[h * 16 + j, pl.ds(q * 16, 16)]
                        rows_v[row, pl.ds(q * 16, 16)] = (
                            av * inv + bv * msk)
                return 0

            lax.fori_loop(0, _CHUNK // 16, sel_body, 0)
            return 0

        lax.fori_loop(0, n_chunks, chunk_body, 0)
        pltpu.async_copy(rows_v, out_hbm.at[pl.ds(wid * b_per_w, b_per_w)],
                         osem).wait()

    return gather_kernel(idx2d, table_a, table_b)


def _tc_select_matmul(ve2, par, w, block_m):
    m, d2 = ve2.shape
    d = d2 // 2
    dm = w.shape[0]

    def mm_body(ve2_ref, par_ref, w_ref, out_ref):
        lo = ve2_ref[:, :d]
        hi = ve2_ref[:, d:]
        ve = jnp.where(par_ref[...] > 0, hi, lo)
        out_ref[...] = lax.dot_general(
            ve,
            w_ref[...],
            (((1,), (1,)), ((), ())),
            preferred_element_type=jnp.float32,
        )

    return pl.pallas_call(
        mm_body,
        grid=(m // block_m,),
        in_specs=[
            pl.BlockSpec((block_m, d2), lambda i: (i, 0)),
            pl.BlockSpec((block_m, 1), lambda i: (i, 0)),
            pl.BlockSpec((dm, d), lambda i: (0, 0)),
        ],
        out_specs=pl.BlockSpec((block_m, dm), lambda i: (i, 0)),
        out_shape=jax.ShapeDtypeStruct((m, dm), jnp.float32),
    )(ve2, par, w)


def kernel(tokens, emb, W):
    b, s = tokens.shape
    m = b * s
    v, d = emb.shape
    dm = W.shape[0]
    n_chunks = m // (_NW * _CHUNK)
    tok = tokens.reshape(m).astype(jnp.int32)
    gidx = (tok // _BV) * _HV + (tok % _HV)
    idx2d = gidx.reshape(_NW * n_chunks, _CHUNK)
    par = ((tok % _BV) // _HV).astype(jnp.float32).reshape(m, 1)
    embT = emb.T
    table_a = _sc_relayout(embT)
    table_b = _tc_relayout(embT, _SC_BLOCKS)
    ve2 = _sc_gather(idx2d, table_a, table_b, n_chunks)
    out = _tc_select_matmul(ve2, par, W, 2048)
    return out.reshape(b, s, dm)


# speedup vs baseline: 4.0976x; 4.0976x over previous
import functools

import jax
import jax.numpy as jnp
from jax import lax
from jax.experimental import pallas as pl
from jax.experimental.pallas import tpu as pltpu
from jax.experimental.pallas import tpu_sc as plsc

_NC = 2
_NS = 16
_NW = _NC * _NS
_CHUNK = 128
_BV = 8192
_QV = _BV // 4


def _tc_relayout_packed(embT):
    d, v = embT.shape
    n_blocks = -(-v // _BV)

    def pack(a, b):
        au = lax.bitcast_convert_type(
            a.astype(jnp.bfloat16), jnp.uint16).astype(jnp.uint32)
        bu = lax.bitcast_convert_type(
            b.astype(jnp.bfloat16), jnp.uint16).astype(jnp.uint32)
        return lax.bitcast_convert_type(au | (bu << 16), jnp.float32)

    def body(in_ref, out_ref):
        x = in_ref[...]
        t = [jnp.transpose(x[:, j * _QV:(j + 1) * _QV], (1, 0))
             for j in range(4)]
        out_ref[:, :d] = pack(t[0], t[1])
        out_ref[:, d:] = pack(t[2], t[3])

    return pl.pallas_call(
        body,
        grid=(n_blocks,),
        in_specs=[pl.BlockSpec((d, _BV), lambda i: (0, i))],
        out_specs=pl.BlockSpec((_QV, 2 * d), lambda i: (i, 0)),
        out_shape=jax.ShapeDtypeStruct((n_blocks * _QV, 2 * d), jnp.float32),
    )(embT)


def _sc_gather(idx2d, table2, n_chunks):
    m = idx2d.shape[0] * idx2d.shape[1]
    d2 = table2.shape[1]
    b_per_w = n_chunks * _CHUNK
    mesh = plsc.VectorSubcoreMesh(core_axis_name="c", subcore_axis_name="s")

    @functools.partial(
        pl.kernel,
        mesh=mesh,
        out_type=jax.ShapeDtypeStruct((m, d2), jnp.float32),
        scratch_types=[
            pltpu.VMEM((n_chunks, _CHUNK), jnp.int32),
            pltpu.VMEM((b_per_w, d2), jnp.float32),
            pltpu.SemaphoreType.DMA,
        ],
    )
    def gather_kernel(idx_hbm, table_hbm, out_hbm, idx_v, rows_v, sem):
        wid = lax.axis_index("s") * _NC + lax.axis_index("c")
        pltpu.sync_copy(idx_hbm.at[pl.ds(wid * n_chunks, n_chunks)], idx_v)
        copies = []
        for j in range(n_chunks):
            copies.append(
                pltpu.async_copy(
                    table_hbm.at[idx_v.at[j]],
                    rows_v.at[pl.ds(j * _CHUNK, _CHUNK)],
                    sem,
                )
            )
        for c in copies:
            c.wait()
        pltpu.sync_copy(rows_v, out_hbm.at[pl.ds(wid * b_per_w, b_per_w)])

    return gather_kernel(idx2d, table2)


def _tc_select_matmul(ve2, mh, ml, w, block_m):
    m, d2 = ve2.shape
    d = d2 // 2
    dm = w.shape[0]

    def mm_body(ve2_ref, mh_ref, ml_ref, w_ref, out_ref):
        x = ve2_ref[...]
        half = jnp.where(mh_ref[...] > 0, x[:, d:], x[:, :d])
        u = lax.bitcast_convert_type(half, jnp.uint32)
        lo_f = lax.bitcast_convert_type(u << 16, jnp.float32)
        hi_f = lax.bitcast_convert_type(u & jnp.uint32(0xFFFF0000),
                                        jnp.float32)
        ve = jnp.where(ml_ref[...] > 0, hi_f, lo_f)
        out_ref[...] = lax.dot_general(
            ve,
            w_ref[...],
            (((1,), (1,)), ((), ())),
            preferred_element_type=jnp.float32,
        )

    return pl.pallas_call(
        mm_body,
        grid=(m // block_m,),
        in_specs=[
            pl.BlockSpec((block_m, d2), lambda i: (i, 0)),
            pl.BlockSpec((block_m, 1), lambda i: (i, 0)),
            pl.BlockSpec((block_m, 1), lambda i: (i, 0)),
            pl.BlockSpec((dm, d), lambda i: (0, 0)),
        ],
        out_specs=pl.BlockSpec((block_m, dm), lambda i: (i, 0)),
        out_shape=jax.ShapeDtypeStruct((m, dm), jnp.float32),
    )(ve2, mh, ml, w)


def kernel(tokens, emb, W):
    b, s = tokens.shape
    m = b * s
    v, d = emb.shape
    dm = W.shape[0]
    n_chunks = m // (_NW * _CHUNK)
    tok = tokens.reshape(m).astype(jnp.int32)
    slot = (tok % _BV) // _QV
    gidx = (tok // _BV) * _QV + (tok % _QV)
    idx2d = gidx.reshape(_NW * n_chunks, _CHUNK)
    mh = (slot // 2).astype(jnp.float32).reshape(m, 1)
    ml = (slot % 2).astype(jnp.float32).reshape(m, 1)
    table2 = _tc_relayout_packed(emb.T)
    ve2 = _sc_gather(idx2d, table2, n_chunks)
    out = _tc_select_matmul(ve2, mh, ml, W, 2048)
    return out.reshape(b, s, dm)


# speedup vs baseline: 4.4693x; 1.0907x over previous
import functools

import jax
import jax.numpy as jnp
from jax import lax
from jax.experimental import pallas as pl
from jax.experimental.pallas import tpu as pltpu
from jax.experimental.pallas import tpu_sc as plsc

_NC = 2
_NS = 16
_NW = _NC * _NS
_CHUNK = 128
_BV = 8192
_QV = _BV // 4


def _tc_relayout_packed(embT):
    d, v = embT.shape
    n_blocks = -(-v // _BV)

    def pack(a, b):
        au = lax.bitcast_convert_type(
            a.astype(jnp.bfloat16), jnp.uint16).astype(jnp.uint32)
        bu = lax.bitcast_convert_type(
            b.astype(jnp.bfloat16), jnp.uint16).astype(jnp.uint32)
        return lax.bitcast_convert_type(au | (bu << 16), jnp.float32)

    def body(in_ref, out_ref):
        x = in_ref[...]
        t = [jnp.transpose(x[:, j * _QV:(j + 1) * _QV], (1, 0))
             for j in range(4)]
        out_ref[:, :d] = pack(t[0], t[1])
        out_ref[:, d:] = pack(t[2], t[3])

    return pl.pallas_call(
        body,
        grid=(n_blocks,),
        in_specs=[pl.BlockSpec((d, _BV), lambda i: (0, i))],
        out_specs=pl.BlockSpec((_QV, 2 * d), lambda i: (i, 0)),
        out_shape=jax.ShapeDtypeStruct((n_blocks * _QV, 2 * d), jnp.float32),
    )(embT)


def _sc_gather(idx2d, table2, n_chunks):
    m = idx2d.shape[0] * idx2d.shape[1]
    d2 = table2.shape[1]
    b_per_w = n_chunks * _CHUNK
    mesh = plsc.VectorSubcoreMesh(core_axis_name="c", subcore_axis_name="s")

    @functools.partial(
        pl.kernel,
        mesh=mesh,
        out_type=jax.ShapeDtypeStruct((m, d2), jnp.float32),
        scratch_types=[
            pltpu.VMEM((n_chunks, _CHUNK), jnp.int32),
            pltpu.VMEM((b_per_w, d2), jnp.float32),
            pltpu.SemaphoreType.DMA,
        ],
    )
    def gather_kernel(idx_hbm, table_hbm, out_hbm, idx_v, rows_v, sem):
        wid = lax.axis_index("s") * _NC + lax.axis_index("c")
        pltpu.sync_copy(idx_hbm.at[pl.ds(wid * n_chunks, n_chunks)], idx_v)
        for g in range(n_chunks):
            for h in range(_CHUNK // 16):
                v = idx_v[g, pl.ds(h * 16, 16)]
                idx_v[g, pl.ds(h * 16, 16)] = (
                    (v >> 13) * _QV + (v & (_QV - 1)))
        copies = []
        for j in range(n_chunks):
            copies.append(
                pltpu.async_copy(
                    table_hbm.at[idx_v.at[j]],
                    rows_v.at[pl.ds(j * _CHUNK, _CHUNK)],
                    sem,
                )
            )
        for c in copies:
            c.wait()
        pltpu.sync_copy(rows_v, out_hbm.at[pl.ds(wid * b_per_w, b_per_w)])

    return gather_kernel(idx2d, table2)


def _tc_select_matmul(ve2, tok2, w, block_m):
    m, d2 = ve2.shape
    d = d2 // 2
    dm = w.shape[0]

    def mm_body(ve2_ref, tok_ref, w_ref, out_ref):
        x = ve2_ref[...]
        slot = (tok_ref[...] & (_BV - 1)) >> 11
        half = jnp.where(slot >= 2, x[:, d:], x[:, :d])
        u = lax.bitcast_convert_type(half, jnp.uint32)
        lo_f = lax.bitcast_convert_type(u << 16, jnp.float32)
        hi_f = lax.bitcast_convert_type(u & jnp.uint32(0xFFFF0000),
                                        jnp.float32)
        ve = jnp.where((slot & 1) > 0, hi_f, lo_f)
        out_ref[...] = lax.dot_general(
            ve,
            w_ref[...],
            (((1,), (1,)), ((), ())),
            preferred_element_type=jnp.float32,
        )

    return pl.pallas_call(
        mm_body,
        grid=(m // block_m,),
        in_specs=[
            pl.BlockSpec((block_m, d2), lambda i: (i, 0)),
            pl.BlockSpec((block_m, 1), lambda i: (i, 0)),
            pl.BlockSpec((dm, d), lambda i: (0, 0)),
        ],
        out_specs=pl.BlockSpec((block_m, dm), lambda i: (i, 0)),
        out_shape=jax.ShapeDtypeStruct((m, dm), jnp.float32),
    )(ve2, tok2, w)


def kernel(tokens, emb, W):
    b, s = tokens.shape
    m = b * s
    v, d = emb.shape
    dm = W.shape[0]
    n_chunks = m // (_NW * _CHUNK)
    tok = tokens.reshape(m).astype(jnp.int32)
    tok2d = tok.reshape(_NW * n_chunks, _CHUNK)
    table2 = _tc_relayout_packed(emb.T)
    ve2 = _sc_gather(tok2d, table2, n_chunks)
    out = _tc_select_matmul(ve2, tok.reshape(m, 1), W, 2048)
    return out.reshape(b, s, dm)
